# Initial kernel scaffold; baseline (speedup 1.0000x reference)
import jax
import jax.numpy as jnp
from jax.experimental import pallas as pl


def kernel(x, edge_index, batch, W1, b1, Wp1, bp1, W2, b2, Wp2, bp2, W3, b3, Wp3, bp3, Wl1, bl1, Wl2, bl2, Wpi, bpi, Wv, bv):
    raise NotImplementedError("write your pallas kernel here")


# speedup vs baseline: 14.6693x; 14.6693x over previous
import functools

import numpy as np
import jax
import jax.numpy as jnp
from jax import lax
from jax.experimental import pallas as pl
from jax.experimental.pallas import tpu as pltpu
from jax.experimental.pallas import tpu_sc as plsc

N = 10000
E = 320000
G = 64
D = 128
NC = 2
NS = 16
NW = NC * NS
EPW = E // NW
CB = 128
NFULL = EPW // CB
CTAIL = EPW - NFULL * CB
NPAD = 10240
RPT = NPAD // NS

_MESH = plsc.VectorSubcoreMesh(core_axis_name="c", subcore_axis_name="s")


def _mp_body(h_hbm, src_hbm, dst_hbm, ew_hbm, dinv_hbm, out_hbm,
             dinv_v, idx_v, dst_v, coef_v, rows_v, idx_t, dst_t, coef_t,
             rows_t, acc_sh):
    c = lax.axis_index("c")
    s = lax.axis_index("s")
    wid = c * NS + s
    ebase = wid * EPW

    def _zrow(r, _):
        for j in range(8):
            rows_v[r, pl.ds(16 * j, 16)] = jnp.zeros((16,), jnp.float32)
        return _
    lax.fori_loop(jnp.int32(0), jnp.int32(CB), _zrow, jnp.int32(0))
    for kk in range(5):
        pltpu.sync_copy(rows_v, acc_sh.at[pl.ds(s * RPT + kk * CB, CB)])

    pltpu.sync_copy(dinv_hbm, dinv_v)
    plsc.subcore_barrier()

    def _chunk(off, cb, idx_r, dst_r, coef_r, rows_r):
        pltpu.sync_copy(src_hbm.at[pl.ds(off, cb)], idx_r)
        pltpu.sync_copy(dst_hbm.at[pl.ds(off, cb)], dst_r)
        pltpu.sync_copy(ew_hbm.at[pl.ds(off, cb)], coef_r)
        def _coef(j, _):
            sl = pl.ds(j * 16, 16)
            sv = idx_r[sl]
            dv = dst_r[sl]
            cv = (plsc.load_gather(dinv_v, [sv]) *
                  plsc.load_gather(dinv_v, [dv]) * coef_r[sl])
            coef_r[sl] = cv
            return _
        lax.fori_loop(jnp.int32(0), jnp.int32(cb // 16), _coef, jnp.int32(0))
        pltpu.sync_copy(h_hbm.at[idx_r], rows_r)
        def _scale(g16, _):
            cv = coef_r[pl.ds(g16 * 16, 16)]
            for r16 in range(16):
                r = g16 * 16 + r16
                cr = cv[r16]
                for j in range(8):
                    sl = pl.ds(16 * j, 16)
                    rows_r[r, sl] = rows_r[r, sl] * cr
            return _
        lax.fori_loop(jnp.int32(0), jnp.int32(cb // 16), _scale, jnp.int32(0))
        pltpu.sync_copy(rows_r, acc_sh.at[dst_r], add=True)

    def _full(i, _):
        _chunk(ebase + i * CB, CB, idx_v, dst_v, coef_v, rows_v)
        return _
    lax.fori_loop(jnp.int32(0), jnp.int32(NFULL), _full, jnp.int32(0))
    _chunk(ebase + NFULL * CB, CTAIL, idx_t, dst_t, coef_t, rows_t)

    plsc.subcore_barrier()
    pltpu.sync_copy(acc_sh.at[pl.ds(s * RPT, RPT)],
                    out_hbm.at[c, pl.ds(s * RPT, RPT)])


_mp_call = pl.kernel(
    _mp_body,
    out_type=jax.ShapeDtypeStruct((NC, NPAD, D), jnp.float32),
    mesh=_MESH,
    compiler_params=pltpu.CompilerParams(needs_layout_passes=False),
    scratch_types=[
        pltpu.VMEM((N,), jnp.float32),
        pltpu.VMEM((CB,), jnp.int32),
        pltpu.VMEM((CB,), jnp.int32),
        pltpu.VMEM((CB,), jnp.float32),
        pltpu.VMEM((CB, D), jnp.float32),
        pltpu.VMEM((CTAIL,), jnp.int32),
        pltpu.VMEM((CTAIL,), jnp.int32),
        pltpu.VMEM((CTAIL,), jnp.float32),
        pltpu.VMEM((CTAIL, D), jnp.float32),
        pltpu.VMEM_SHARED((NPAD, D), jnp.float32),
    ],
)


def _scalar_pass_body(make_val, writeback):
    def body(src_hbm, dst_hbm, ew_hbm, tab1_hbm, tab2_hbm, out_hbm, ew_out,
             tab1_v, tab2_v, src_c, dst_c, ew_c, rows_c, acc_sh):
        c = lax.axis_index("c")
        s = lax.axis_index("s")
        wid = c * NS + s
        ebase = wid * EPW
        lane0 = lax.iota(jnp.int32, 16) == 0

        def _zrow(r, carry):
            rows_c[r, :] = jnp.zeros((16,), jnp.float32)
            return carry
        lax.fori_loop(jnp.int32(0), jnp.int32(CB), _zrow, jnp.int32(0))
        for kk in range(5):
            pltpu.sync_copy(rows_c, acc_sh.at[pl.ds(s * RPT + kk * CB, CB)])
        pltpu.sync_copy(tab1_hbm, tab1_v)
        pltpu.sync_copy(tab2_hbm, tab2_v)
        plsc.subcore_barrier()

        def _chunk(off, cb):
            pltpu.sync_copy(src_hbm.at[pl.ds(off, cb)], src_c.at[pl.ds(0, cb)])
            pltpu.sync_copy(dst_hbm.at[pl.ds(off, cb)], dst_c.at[pl.ds(0, cb)])
            pltpu.sync_copy(ew_hbm.at[pl.ds(off, cb)], ew_c.at[pl.ds(0, cb)])

            def _grp(j, carry):
                sl = pl.ds(j * 16, 16)
                val = make_val(tab1_v, tab2_v, src_c[sl], dst_c[sl], ew_c[sl])
                if writeback:
                    ew_c[sl] = val
                for r16 in range(16):
                    rows_c[j * 16 + r16, :] = jnp.where(
                        lane0, val[r16], jnp.float32(0.0))
                return carry
            lax.fori_loop(jnp.int32(0), jnp.int32(cb // 16), _grp, jnp.int32(0))
            pltpu.sync_copy(rows_c.at[pl.ds(0, cb)],
                            acc_sh.at[dst_c.at[pl.ds(0, cb)]] if cb != CB
                            else acc_sh.at[dst_c], add=True)
            if writeback:
                pltpu.sync_copy(ew_c.at[pl.ds(0, cb)],
                                ew_out.at[pl.ds(off, cb)])

        def _full(i, carry):
            _chunk(ebase + i * CB, CB)
            return carry
        lax.fori_loop(jnp.int32(0), jnp.int32(NFULL), _full, jnp.int32(0))
        _chunk(ebase + NFULL * CB, CTAIL)

        plsc.subcore_barrier()
        pltpu.sync_copy(acc_sh.at[pl.ds(s * RPT, RPT)],
                        out_hbm.at[c, pl.ds(s * RPT, RPT)])

    return body


def _deg_val(kf_v, _unused_v, sv, dv, ewv):
    return ewv * plsc.load_gather(kf_v, [sv]) * plsc.load_gather(kf_v, [dv])


def _score_val(dinv_v, sh_v, sv, dv, ewv):
    return (plsc.load_gather(dinv_v, [sv]) * plsc.load_gather(dinv_v, [dv])
            * ewv * plsc.load_gather(sh_v, [sv]))


def _make_scalar_pass(make_val, writeback):
    outs = [jax.ShapeDtypeStruct((NC, NPAD, 16), jnp.float32)]
    if writeback:
        outs.append(jax.ShapeDtypeStruct((E,), jnp.float32))

    body = _scalar_pass_body(make_val, writeback)

    def full_body(src_hbm, dst_hbm, ew_hbm, t1, t2, out_hbm, *rest):
        if writeback:
            ew_out = rest[0]
            scratch = rest[1:]
        else:
            ew_out = None
            scratch = rest
        body(src_hbm, dst_hbm, ew_hbm, t1, t2, out_hbm, ew_out, *scratch)

    return pl.kernel(
        full_body,
        out_type=tuple(outs) if writeback else outs[0],
        mesh=_MESH,
        compiler_params=pltpu.CompilerParams(needs_layout_passes=False),
        scratch_types=[
            pltpu.VMEM((N,), jnp.float32),
            pltpu.VMEM((N,), jnp.float32),
            pltpu.VMEM((CB,), jnp.int32),
            pltpu.VMEM((CB,), jnp.int32),
            pltpu.VMEM((CB,), jnp.float32),
            pltpu.VMEM((CB, 16), jnp.float32),
            pltpu.VMEM_SHARED((NPAD, 16), jnp.float32),
        ],
    )


_deg_call = _make_scalar_pass(_deg_val, True)
_scedge_call = _make_scalar_pass(_score_val, False)


def _deg_ew_pass(src, dst, ew, kf):
    parts, ew_new = _deg_call(src, dst, ew, kf, kf)
    deg = (parts[0, :, 0] + parts[1, :, 0])[:N] + 1.0
    return deg, ew_new


def _score_pass(src, dst, ew, dinv, s_h):
    parts = _scedge_call(src, dst, ew, dinv, s_h)
    return (parts[0, :, 0] + parts[1, :, 0])[:N]


def _message_pass(h, src, dst, ew, dinv):
    parts = _mp_call(h, src, dst, ew, dinv)
    return (parts[0] + parts[1])[:N]


NT = 79
I0 = np.int32(0)
NP = NT * 128

F32 = jnp.float32


def _bcast_col(row):
    return jnp.transpose(jnp.broadcast_to(row, (128, 128)))


def _mm_body(x_ref, w_ref, degsum_ref, h_ref, dinv_ref):
    h_ref[...] = jnp.dot(x_ref[...], w_ref[...],
                         preferred_element_type=F32)
    dinv_ref[...] = lax.rsqrt(degsum_ref[...] + jnp.float32(1.0))


def _mm_call(xp, W, degsum3):
    return pl.pallas_call(
        _mm_body,
        grid=(NT,),
        in_specs=[
            pl.BlockSpec((128, 128), lambda t: (t, I0)),
            pl.BlockSpec((128, 128), lambda t: (I0, I0)),
            pl.BlockSpec((1, 1, 128), lambda t: (t, I0, I0)),
        ],
        out_specs=[
            pl.BlockSpec((128, 128), lambda t: (t, I0)),
            pl.BlockSpec((1, 1, 128), lambda t: (t, I0, I0)),
        ],
        out_shape=[
            jax.ShapeDtypeStruct((NP, 128), F32),
            jax.ShapeDtypeStruct((NT, 1, 128), F32),
        ],
    )(xp, W, degsum3)


def _combine_body(agg_ref, h_ref, dinv_ref, b_ref, wpt_ref, g_ref, sh_ref):
    dinv = jnp.reshape(dinv_ref[...], (1, 128))
    invdeg_col = _bcast_col(dinv * dinv)
    g = (agg_ref[0] + agg_ref[1] + h_ref[...] * invdeg_col
         + jnp.broadcast_to(b_ref[...], (128, 128)))
    g = jnp.maximum(g, jnp.float32(0.0))
    g_ref[...] = g
    gt = jnp.transpose(g)
    sh_ref[...] = jnp.reshape(
        jnp.dot(wpt_ref[...], gt, preferred_element_type=F32), (1, 1, 128))


def _combine_call(agg, h, dinv3, b_row, wp_t):
    return pl.pallas_call(
        _combine_body,
        grid=(NT,),
        in_specs=[
            pl.BlockSpec((2, 128, 128), lambda t: (I0, t, I0)),
            pl.BlockSpec((128, 128), lambda t: (t, I0)),
            pl.BlockSpec((1, 1, 128), lambda t: (t, I0, I0)),
            pl.BlockSpec((1, 128), lambda t: (I0, I0)),
            pl.BlockSpec((1, 128), lambda t: (I0, I0)),
        ],
        out_specs=[
            pl.BlockSpec((128, 128), lambda t: (t, I0)),
            pl.BlockSpec((1, 1, 128), lambda t: (t, I0, I0)),
        ],
        out_shape=[
            jax.ShapeDtypeStruct((NP, 128), F32),
            jax.ShapeDtypeStruct((NT, 1, 128), F32),
        ],
    )(agg, h, dinv3, b_row, wp_t)


def _score_body(aggs_ref, sh_ref, dinv_ref, alive_ref, bp_ref,
                score_ref, scm_ref):
    dinv = dinv_ref[...]
    score = aggs_ref[...] + sh_ref[...] * dinv * dinv + bp_ref[0]
    score_ref[...] = score
    scm_ref[...] = jnp.where(alive_ref[...] > jnp.float32(0.0), score, jnp.float32(-1e30))


def _score_call(aggs3, sh3, dinv3, alive3, bp):
    blk = pl.BlockSpec((1, 1, 128), lambda t: (t, I0, I0))
    return pl.pallas_call(
        _score_body,
        grid=(NT,),
        in_specs=[blk, blk, blk, blk,
                  pl.BlockSpec((1,), lambda t: (I0,),
                               memory_space=pltpu.SMEM)],
        out_specs=[blk, blk],
        out_shape=[
            jax.ShapeDtypeStruct((NT, 1, 128), F32),
            jax.ShapeDtypeStruct((NT, 1, 128), F32),
        ],
    )(aggs3, sh3, dinv3, alive3, bp)


def _rankread_body(lo_ref, hi_ref, bf_ref, bl_ref,
                   scm_all, batf_all, score_ref, alive_ref, kvec_ref, g_ref,
                   xnew_ref, keep_ref, mx_ref, sm_ref, cnt_ref):
    t = pl.program_id(0)

    @pl.when(t == 0)
    def _init():
        mx_ref[...] = jnp.full((G, 128), -1e30, F32)
        sm_ref[...] = jnp.zeros((G, 128), F32)
        cnt_ref[...] = jnp.zeros((G, 1), F32)

    scm_i = jnp.reshape(scm_all[pl.ds(t, 1)], (1, 128))
    batf_i = jnp.reshape(batf_all[pl.ds(t, 1)], (1, 128))
    si_rows = jnp.broadcast_to(scm_i, (128, 128))
    bi_rows = jnp.broadcast_to(batf_i, (128, 128))
    ig = jnp.broadcast_to(
        lax.broadcasted_iota(jnp.int32, (1, 128), 1).astype(F32)
        + lax.convert_element_type(t, F32) * jnp.float32(128.0), (128, 128))

    def jbody(jt, acc):
        scm_j = jnp.reshape(scm_all[pl.ds(jt, 1)], (1, 128))
        batf_j = jnp.reshape(batf_all[pl.ds(jt, 1)], (1, 128))
        sj = _bcast_col(scm_j)
        bj = _bcast_col(batf_j)
        jg = (lax.broadcasted_iota(jnp.int32, (128, 128), 0).astype(F32)
              + lax.convert_element_type(jt, F32) * jnp.float32(128.0))
        cmp = (bj == bi_rows) & ((sj > si_rows)
                                 | ((sj == si_rows) & (jg < ig)))
        return acc + jnp.sum(cmp.astype(F32), axis=0, keepdims=True)

    rank = lax.fori_loop(lo_ref[t], hi_ref[t], jbody,
                         jnp.zeros((1, 128), F32))

    onehot = (lax.broadcasted_iota(jnp.int32, (G, 128), 0).astype(F32)
              == jnp.broadcast_to(batf_i, (G, 128))).astype(F32)
    k_i = jnp.dot(kvec_ref[...], onehot, preferred_element_type=F32)
    alive_i = jnp.reshape(alive_ref[...], (1, 128))
    keepf = ((rank < k_i) & (alive_i > jnp.float32(0.0))).astype(F32)
    keep_ref[...] = jnp.reshape(keepf, (1, 1, 128))

    score_i = jnp.reshape(score_ref[...], (1, 128))
    kcol = _bcast_col(keepf)
    xnew = g_ref[...] * jnp.tanh(_bcast_col(score_i)) * kcol
    xnew_ref[...] = xnew

    okeep = onehot * jnp.broadcast_to(keepf, (G, 128))
    sm_ref[...] += jnp.dot(okeep, xnew, preferred_element_type=F32)
    cnt_ref[...] += jnp.sum(okeep, axis=1, keepdims=True)

    bcol = _bcast_col(batf_i)
    bf = bf_ref[t]
    bl = bl_ref[t]
    for g in range(G):
        @pl.when((g >= bf) & (g <= bl))
        def _upd():
            mask = (bcol == jnp.float32(g)) & (kcol > jnp.float32(0.0))
            m = jnp.max(jnp.where(mask, xnew, jnp.float32(-1e30)), axis=0, keepdims=True)
            mx_ref[g:g + 1, :] = jnp.maximum(mx_ref[g:g + 1, :], m)


def _rankread_call(lo, hi, bf, bl, scm3, batf3, score3, alive3, kvec, gfeat):
    blk = pl.BlockSpec((1, 1, 128), lambda t, *_: (t, I0, I0))
    whole3 = pl.BlockSpec((NT, 1, 128), lambda t, *_: (I0, I0, I0))
    grid_spec = pltpu.PrefetchScalarGridSpec(
        num_scalar_prefetch=4,
        grid=(NT,),
        in_specs=[whole3, whole3, blk, blk,
                  pl.BlockSpec((1, G), lambda t, *_: (I0, I0)),
                  pl.BlockSpec((128, 128), lambda t, *_: (t, I0))],
        out_specs=[
            pl.BlockSpec((128, 128), lambda t, *_: (t, I0)),
            blk,
            pl.BlockSpec((G, 128), lambda t, *_: (I0, I0)),
            pl.BlockSpec((G, 128), lambda t, *_: (I0, I0)),
            pl.BlockSpec((G, 1), lambda t, *_: (I0, I0)),
        ],
    )
    return pl.pallas_call(
        _rankread_body,
        grid_spec=grid_spec,
        out_shape=[
            jax.ShapeDtypeStruct((NP, 128), F32),
            jax.ShapeDtypeStruct((NT, 1, 128), F32),
            jax.ShapeDtypeStruct((G, 128), F32),
            jax.ShapeDtypeStruct((G, 128), F32),
            jax.ShapeDtypeStruct((G, 1), F32),
        ],
    )(lo, hi, bf, bl, scm3, batf3, score3, alive3, kvec, gfeat)


def _final_body(mx1, sm1, c1, mx2, sm2, c2, mx3, sm3, c3,
                wl1, bl1r, wl2, bl2r, wpi, bpir, wv, bvs, pi_ref, v_ref):
    def xr(mx, sm, c):
        mean = sm[...] / jnp.maximum(c[...], jnp.float32(1.0))
        return jnp.concatenate([mx[...], mean], axis=1)

    z = xr(mx1, sm1, c1) + xr(mx2, sm2, c2) + xr(mx3, sm3, c3)
    z = jnp.maximum(jnp.dot(z, wl1[...], preferred_element_type=F32)
                    + jnp.broadcast_to(bl1r[...], (G, 128)), jnp.float32(0.0))
    z = jnp.maximum(jnp.dot(z, wl2[...], preferred_element_type=F32)
                    + jnp.broadcast_to(bl2r[...], (G, 64)), jnp.float32(0.0))
    logits = (jnp.dot(z, wpi[...], preferred_element_type=F32)
              + jnp.broadcast_to(bpir[...], (G, 512)))
    m = jnp.max(logits, axis=1, keepdims=True)
    s = jnp.sum(jnp.exp(logits - m), axis=1, keepdims=True)
    pi_ref[...] = logits - m - jnp.log(s)
    v_ref[...] = jnp.maximum(
        jnp.dot(z, wv[...], preferred_element_type=F32) + bvs[0], jnp.float32(0.0))


def _final_call(r1, r2, r3, Wl1, bl1, Wl2, bl2, Wpi, bpi, Wv, bv):
    vm = pl.BlockSpec(memory_space=pltpu.VMEM)
    sm = pl.BlockSpec(memory_space=pltpu.SMEM)
    ins = [vm] * 16 + [sm]
    return pl.pallas_call(
        _final_body,
        in_specs=ins,
        out_specs=[vm, vm],
        out_shape=[
            jax.ShapeDtypeStruct((G, 512), F32),
            jax.ShapeDtypeStruct((G, 1), F32),
        ],
    )(*r1, *r2, *r3, Wl1, bl1.reshape(1, 128), Wl2, bl2.reshape(1, 64),
      Wpi, bpi.reshape(1, 512), Wv, bv)


def kernel(x, edge_index, batch, W1, b1, Wp1, bp1, W2, b2, Wp2, bp2,
           W3, b3, Wp3, bp3, Wl1, bl1, Wl2, bl2, Wpi, bpi, Wv, bv):
    x = x.astype(F32)
    src = edge_index[0].astype(jnp.int32)
    dst = edge_index[1].astype(jnp.int32)
    batch = batch.astype(jnp.int32)

    xp = jnp.pad(x, ((0, NP - N), (0, 0)))
    batp = jnp.concatenate([batch, jnp.full((NP - N,), G, jnp.int32)])
    batf3 = batp.astype(F32).reshape(NT, 1, 128)
    alive3 = (jnp.arange(NP) < N).astype(F32).reshape(NT, 1, 128)

    gr = jnp.arange(G)
    seg_start = jnp.searchsorted(batch, gr).astype(jnp.int32)
    seg_end = jnp.searchsorted(batch, gr + 1).astype(jnp.int32)
    counts = (seg_end - seg_start).astype(F32)
    bfirst = batch[jnp.arange(NT) * 128]
    blast = batch[jnp.minimum(jnp.arange(NT) * 128 + 127, N - 1)]
    lo = (seg_start[bfirst] // 128).astype(jnp.int32)
    hi = ((seg_end[blast] + 127) // 128).astype(jnp.int32)
    bfirst = bfirst.astype(jnp.int32)
    blast = blast.astype(jnp.int32)

    ew = jnp.ones((E,), F32)
    kf = jnp.ones((N,), F32)
    kvec = jnp.ceil(0.5 * counts).reshape(1, G)

    h = xp
    reads = []
    for (W, b, Wp, bp) in ((W1, b1, Wp1, bp1), (W2, b2, Wp2, bp2),
                           (W3, b3, Wp3, bp3)):
        deg_parts, ew = _deg_call(src, dst, ew, kf, kf)
        degsum3 = jnp.pad((deg_parts[0, :, 0] + deg_parts[1, :, 0])[:N],
                          (0, NP - N)).reshape(NT, 1, 128)
        h_mm, dinv3 = _mm_call(h, W, degsum3)
        dinv_flat = dinv3.reshape(NP)[:N]
        agg = _mp_call(h_mm, src, dst, ew, dinv_flat)
        g, sh3 = _combine_call(agg, h_mm, dinv3, b.reshape(1, 128),
                               Wp.reshape(1, 128))
        sh_flat = sh3.reshape(NP)[:N]
        aggs_parts = _scedge_call(src, dst, ew, dinv_flat, sh_flat)
        aggs3 = jnp.pad((aggs_parts[0, :, 0] + aggs_parts[1, :, 0])[:N],
                        (0, NP - N)).reshape(NT, 1, 128)
        score3, scm3 = _score_call(aggs3, sh3, dinv3, alive3, bp)
        h, keep3, mx, sm, cnt = _rankread_call(
            lo, hi, bfirst, blast, scm3, batf3, score3, alive3, kvec, g)
        kf = keep3.reshape(NP)[:N]
        alive3 = keep3
        kvec = jnp.ceil(0.5 * cnt).reshape(1, G)
        reads.append((mx, sm, cnt))

    pi, v = _final_call(reads[0], reads[1], reads[2],
                        Wl1, bl1, Wl2, bl2, Wpi, bpi, Wv, bv)
    return pi, v
